# baseline (speedup 1.0000x reference)
import functools

import jax
import jax.numpy as jnp
from jax import lax
from jax.experimental import pallas as pl
from jax.experimental.pallas import tpu as pltpu
from jax.experimental.pallas import tpu_sc as plsc

N = 10000
E = 320000
D = 128
H = 128
O = 64
G = 128

NC = 2
NS = 16
L = 128

NP = 10240
KW = 80
CH = 40
EPAD = NC * NS * KW * L


def _deg_sc_body(ones_hbm, dst_hbm, zeros_hbm, out_hbm,
                 dst_v, ones_v, deg_s):
    c = lax.axis_index("c")
    s = lax.axis_index("s")
    apt = NP // NS
    pltpu.sync_copy(ones_hbm, ones_v)
    pltpu.sync_copy(zeros_hbm.at[pl.ds(s * apt, apt)], deg_s.at[pl.ds(s * apt, apt)])
    plsc.subcore_barrier()

    def chunk(q, carry):
        pltpu.sync_copy(dst_hbm.at[c, s, pl.ds(q * CH, CH)], dst_v)

        def body(j, carry2):
            pltpu.sync_copy(ones_v, deg_s.at[dst_v.at[j]], add=True)
            return carry2

        lax.fori_loop(0, CH, body, 0)
        return carry

    lax.fori_loop(0, KW // CH, chunk, 0)
    plsc.subcore_barrier()
    pltpu.sync_copy(deg_s.at[pl.ds(s * apt, apt)], out_hbm.at[c, pl.ds(s * apt, apt)])

def _agg_sc_body(y_hbm, src_hbm, dst_hbm, zeros_hbm, out_hbm,
                 src_v, dst_v, rows_a, rows_b, agg_s,
                 sem_a, sem_b, sem_sa, sem_sb):
    c = lax.axis_index("c")
    s = lax.axis_index("s")
    apt = NP // NS
    pltpu.sync_copy(zeros_hbm.at[pl.ds(s * apt, apt)], agg_s.at[pl.ds(s * apt, apt)])
    plsc.subcore_barrier()

    def chunk(q, carry):
        pltpu.sync_copy(src_hbm.at[c, s, pl.ds(q * CH, CH)], src_v)
        pltpu.sync_copy(dst_hbm.at[c, s, pl.ds(q * CH, CH)], dst_v)
        pltpu.async_copy(y_hbm.at[src_v.at[0]], rows_a, sem_a)

        def body(t, carry2):
            j0 = 2 * t
            j1 = j0 + 1
            pltpu.make_async_copy(y_hbm.at[src_v.at[j0]], rows_a, sem_a).wait()
            pltpu.async_copy(y_hbm.at[src_v.at[j1]], rows_b, sem_b)
            pltpu.sync_copy(rows_a, agg_s.at[dst_v.at[j0]], add=True)
            pltpu.make_async_copy(y_hbm.at[src_v.at[j1]], rows_b, sem_b).wait()

            @pl.when(t < CH // 2 - 1)
            def _():
                pltpu.async_copy(y_hbm.at[src_v.at[j0 + 2]], rows_a, sem_a)

            pltpu.sync_copy(rows_b, agg_s.at[dst_v.at[j1]], add=True)
            return carry2

        lax.fori_loop(0, CH // 2, body, 0)
        return carry

    lax.fori_loop(0, KW // CH, chunk, 0)
    plsc.subcore_barrier()
    pltpu.sync_copy(agg_s.at[pl.ds(s * apt, apt)], out_hbm.at[c, pl.ds(s * apt, apt)])


@functools.cache
def _sc_kernels():
    mesh = plsc.VectorSubcoreMesh(core_axis_name="c", subcore_axis_name="s")
    deg = pl.kernel(
        _deg_sc_body,
        out_type=jax.ShapeDtypeStruct((NC, NP, H), jnp.float32),
        mesh=mesh,
        scratch_types=[
            pltpu.VMEM((CH, L), jnp.int32),
            pltpu.VMEM((L, H), jnp.float32),
            pltpu.VMEM_SHARED((NP, H), jnp.float32),
        ],
    )
    agg = pl.kernel(
        _agg_sc_body,
        out_type=jax.ShapeDtypeStruct((NC, NP, H), jnp.float32),
        mesh=mesh,
        scratch_types=[
            pltpu.VMEM((CH, L), jnp.int32),
            pltpu.VMEM((CH, L), jnp.int32),
            pltpu.VMEM((L, H), jnp.float32),
            pltpu.VMEM((L, H), jnp.float32),
            pltpu.VMEM_SHARED((NP, H), jnp.float32),
            pltpu.SemaphoreType.DMA,
            pltpu.SemaphoreType.DMA,
            pltpu.SemaphoreType.DMA,
            pltpu.SemaphoreType.DMA,
        ],
    )
    return deg, agg


def _tcA_body(x_ref, w_ref, degp_ref, y_ref, dis_ref):
    deg = degp_ref[0, :N, 0:1] + degp_ref[1, :N, 0:1] + 1.0
    dis = lax.rsqrt(deg)
    dis_ref[...] = dis
    xw = jnp.dot(x_ref[...], w_ref[...], preferred_element_type=jnp.float32)
    y_ref[...] = xw * dis


def _bn_relu(agg_ref, y_ref, dis_ref, b_ref, g_ref, be_ref):
    dis = dis_ref[...]
    a = agg_ref[0, :N, :] + agg_ref[1, :N, :] + y_ref[...]
    z = a * dis + b_ref[...]
    mu = jnp.mean(z, axis=0, keepdims=True)
    zc = z - mu
    var = jnp.mean(zc * zc, axis=0, keepdims=True)
    h = jnp.maximum(zc * lax.rsqrt(var + 1e-5) * g_ref[...] + be_ref[...], 0.0)
    return h, dis


def _tcB_body(agg_ref, y_ref, dis_ref, b_ref, g_ref, be_ref, w_ref, out_ref):
    h, dis = _bn_relu(agg_ref, y_ref, dis_ref, b_ref, g_ref, be_ref)
    out_ref[...] = jnp.dot(h, w_ref[...], preferred_element_type=jnp.float32) * dis


def _tcC_body(agg_ref, y_ref, dis_ref, b_ref, g_ref, be_ref,
              batch_ref, fcw_ref, fcb_ref, out_ref):
    h, _ = _bn_relu(agg_ref, y_ref, dis_ref, b_ref, g_ref, be_ref)
    grp = lax.broadcasted_iota(jnp.int32, (G, N), 0)
    onehot = (batch_ref[...] == grp).astype(jnp.float32)
    ssum = jnp.dot(onehot, h, preferred_element_type=jnp.float32)
    cnt = jnp.sum(onehot, axis=1, keepdims=True)
    pooled = ssum / jnp.maximum(cnt, 1.0)
    out_ref[...] = jnp.dot(pooled, fcw_ref[...],
                           preferred_element_type=jnp.float32) + fcb_ref[...]


_tcA = pl.pallas_call(
    _tcA_body, out_shape=(jax.ShapeDtypeStruct((N, H), jnp.float32),
                          jax.ShapeDtypeStruct((N, 1), jnp.float32)))
_tcB = pl.pallas_call(
    _tcB_body, out_shape=jax.ShapeDtypeStruct((N, H), jnp.float32))
_tcC = pl.pallas_call(
    _tcC_body, out_shape=jax.ShapeDtypeStruct((G, O), jnp.float32))


@jax.jit
def kernel(x, edge_index, batch,
           W1, b1, g1, be1, W2, b2, g2, be2, W3, b3, g3, be3, fcW, fcb):
    src = edge_index[0]
    dst = edge_index[1]
    pad = EPAD - E
    padi = jnp.arange(pad, dtype=jnp.int32)
    srcp = jnp.concatenate([src, padi % 16])
    dstp = jnp.concatenate([dst, N + (padi % 128)])
    src_a = srcp.reshape(NC, NS, KW, L)
    dst_a = dstp.reshape(NC, NS, KW, L)

    zerosH = jnp.zeros((NP, H), jnp.float32)
    onesL = jnp.ones((L, H), jnp.float32)

    _deg_sc, _agg_sc = _sc_kernels()
    degp = _deg_sc(onesL, dst_a, zerosH)

    b1r, g1r, be1r = b1.reshape(1, H), g1.reshape(1, H), be1.reshape(1, H)
    b2r, g2r, be2r = b2.reshape(1, H), g2.reshape(1, H), be2.reshape(1, H)
    b3r, g3r, be3r = b3.reshape(1, H), g3.reshape(1, H), be3.reshape(1, H)

    y1, dis = _tcA(x, W1, degp)
    a1 = _agg_sc(y1, src_a, dst_a, zerosH)
    y2 = _tcB(a1, y1, dis, b1r, g1r, be1r, W2)
    a2 = _agg_sc(y2, src_a, dst_a, zerosH)
    y3 = _tcB(a2, y2, dis, b2r, g2r, be2r, W3)
    a3 = _agg_sc(y3, src_a, dst_a, zerosH)
    out = _tcC(a3, y3, dis, b3r, g3r, be3r,
               batch.reshape(1, N), fcW, fcb.reshape(1, O))
    return out


# speedup vs baseline: 1.0275x; 1.0275x over previous
import functools

import jax
import jax.numpy as jnp
from jax import lax
from jax.experimental import pallas as pl
from jax.experimental.pallas import tpu as pltpu
from jax.experimental.pallas import tpu_sc as plsc

N = 10000
E = 320000
D = 128
H = 128
O = 64
G = 128

NC = 2
NS = 16
L = 128

NP = 10240
KW = 80
CH = 40
EPAD = NC * NS * KW * L


def _deg_sc_body(ones_hbm, dst_hbm, zeros_hbm, out_hbm,
                 dst_v, ones_v, deg_s):
    c = lax.axis_index("c")
    s = lax.axis_index("s")
    apt = NP // NS
    pltpu.sync_copy(ones_hbm, ones_v)
    pltpu.sync_copy(zeros_hbm.at[pl.ds(s * apt, apt)], deg_s.at[pl.ds(s * apt, apt)])
    plsc.subcore_barrier()

    def chunk(q, carry):
        pltpu.sync_copy(dst_hbm.at[c, s, pl.ds(q * CH, CH)], dst_v)

        def body(j, carry2):
            pltpu.sync_copy(ones_v, deg_s.at[dst_v.at[j]], add=True)
            return carry2

        lax.fori_loop(0, CH, body, 0)
        return carry

    lax.fori_loop(0, KW // CH, chunk, 0)
    plsc.subcore_barrier()
    pltpu.sync_copy(deg_s.at[pl.ds(s * apt, apt)], out_hbm.at[c, pl.ds(s * apt, apt)])

def _agg_sc_body(y_hbm, src_hbm, dst_hbm, zeros_hbm, out_hbm,
                 src_v, dst_v, rows_a, rows_b, agg_s,
                 sem_a, sem_b, sem_sa, sem_sb):
    c = lax.axis_index("c")
    s = lax.axis_index("s")
    apt = NP // NS
    pltpu.sync_copy(zeros_hbm.at[pl.ds(s * apt, apt)], agg_s.at[pl.ds(s * apt, apt)])
    plsc.subcore_barrier()

    def chunk(q, carry):
        pltpu.sync_copy(src_hbm.at[c, s, pl.ds(q * CH, CH)], src_v)
        pltpu.sync_copy(dst_hbm.at[c, s, pl.ds(q * CH, CH)], dst_v)
        pltpu.async_copy(y_hbm.at[src_v.at[0]], rows_a, sem_a)

        def body(t, carry2):
            j0 = 2 * t
            j1 = j0 + 1
            pltpu.make_async_copy(y_hbm.at[src_v.at[j0]], rows_a, sem_a).wait()
            pltpu.async_copy(y_hbm.at[src_v.at[j1]], rows_b, sem_b)
            pltpu.sync_copy(rows_a, agg_s.at[dst_v.at[j0]], add=True)
            pltpu.make_async_copy(y_hbm.at[src_v.at[j1]], rows_b, sem_b).wait()

            @pl.when(t < CH // 2 - 1)
            def _():
                pltpu.async_copy(y_hbm.at[src_v.at[j0 + 2]], rows_a, sem_a)

            pltpu.sync_copy(rows_b, agg_s.at[dst_v.at[j1]], add=True)
            return carry2

        lax.fori_loop(0, CH // 2, body, 0)
        return carry

    lax.fori_loop(0, KW // CH, chunk, 0)
    plsc.subcore_barrier()
    pltpu.sync_copy(agg_s.at[pl.ds(s * apt, apt)], out_hbm.at[c, pl.ds(s * apt, apt)])


@functools.cache
def _sc_kernels():
    mesh = plsc.VectorSubcoreMesh(core_axis_name="c", subcore_axis_name="s")
    deg = pl.kernel(
        _deg_sc_body,
        out_type=jax.ShapeDtypeStruct((NC, NP, H), jnp.float32),
        mesh=mesh,
        scratch_types=[
            pltpu.VMEM((CH, L), jnp.int32),
            pltpu.VMEM((L, H), jnp.float32),
            pltpu.VMEM_SHARED((NP, H), jnp.float32),
        ],
    )
    agg = pl.kernel(
        _agg_sc_body,
        out_type=jax.ShapeDtypeStruct((NC, NP, H), jnp.float32),
        mesh=mesh,
        scratch_types=[
            pltpu.VMEM((CH, L), jnp.int32),
            pltpu.VMEM((CH, L), jnp.int32),
            pltpu.VMEM((L, H), jnp.float32),
            pltpu.VMEM((L, H), jnp.float32),
            pltpu.VMEM_SHARED((NP, H), jnp.float32),
            pltpu.SemaphoreType.DMA,
            pltpu.SemaphoreType.DMA,
            pltpu.SemaphoreType.DMA,
            pltpu.SemaphoreType.DMA,
        ],
    )
    return deg, agg


def _tcA_body(x_ref, w_ref, degp_ref, y_ref, dis_ref):
    deg = degp_ref[0, :N, 0:1] + degp_ref[1, :N, 0:1] + 1.0
    dis = lax.rsqrt(deg)
    dis_ref[:N, :] = dis
    dis_ref[N:, :] = jnp.ones((NP - N, 1), jnp.float32)
    xw = jnp.dot(x_ref[...], w_ref[...], preferred_element_type=jnp.float32)
    y_ref[:N, :] = xw * dis
    y_ref[N:, :] = jnp.zeros((NP - N, H), jnp.float32)


def _bn_relu(agg_ref, y_ref, dis_ref, b_ref, g_ref, be_ref):
    dis = dis_ref[:N, :]
    a = agg_ref[0, :N, :] + agg_ref[1, :N, :] + y_ref[:N, :]
    z = a * dis + b_ref[...]
    mu = jnp.mean(z, axis=0, keepdims=True)
    zc = z - mu
    var = jnp.mean(zc * zc, axis=0, keepdims=True)
    h = jnp.maximum(zc * lax.rsqrt(var + 1e-5) * g_ref[...] + be_ref[...], 0.0)
    return h, dis


def _tcB_body(agg_ref, y_ref, dis_ref, b_ref, g_ref, be_ref, w_ref, out_ref):
    h, dis = _bn_relu(agg_ref, y_ref, dis_ref, b_ref, g_ref, be_ref)
    out_ref[:N, :] = jnp.dot(h, w_ref[...], preferred_element_type=jnp.float32) * dis
    out_ref[N:, :] = jnp.zeros((NP - N, H), jnp.float32)


def _tcC_body(agg_ref, y_ref, dis_ref, b_ref, g_ref, be_ref,
              batch_ref, fcw_ref, fcb_ref, out_ref):
    h, _ = _bn_relu(agg_ref, y_ref, dis_ref, b_ref, g_ref, be_ref)
    grp = lax.broadcasted_iota(jnp.int32, (G, N), 0)
    onehot = (batch_ref[...] == grp).astype(jnp.float32)
    ssum = jnp.dot(onehot, h, preferred_element_type=jnp.float32)
    cnt = jnp.sum(onehot, axis=1, keepdims=True)
    pooled = ssum / jnp.maximum(cnt, 1.0)
    out_ref[...] = jnp.dot(pooled, fcw_ref[...],
                           preferred_element_type=jnp.float32) + fcb_ref[...]


_tcA = pl.pallas_call(
    _tcA_body, out_shape=(jax.ShapeDtypeStruct((NP, H), jnp.float32),
                          jax.ShapeDtypeStruct((NP, 1), jnp.float32)))
_tcB = pl.pallas_call(
    _tcB_body, out_shape=jax.ShapeDtypeStruct((NP, H), jnp.float32))
_tcC = pl.pallas_call(
    _tcC_body, out_shape=jax.ShapeDtypeStruct((G, O), jnp.float32))


@jax.jit
def kernel(x, edge_index, batch,
           W1, b1, g1, be1, W2, b2, g2, be2, W3, b3, g3, be3, fcW, fcb):
    src = edge_index[0]
    dst = edge_index[1]
    pad = EPAD - E
    padi = jnp.arange(pad, dtype=jnp.int32)
    srcp = jnp.concatenate([src, padi % 16])
    dstp = jnp.concatenate([dst, N + (padi % 128)])
    src_a = srcp.reshape(NC, NS, KW, L)
    dst_a = dstp.reshape(NC, NS, KW, L)

    zerosH = jnp.zeros((NP, H), jnp.float32)
    onesL = jnp.ones((L, H), jnp.float32)

    _deg_sc, _agg_sc = _sc_kernels()
    degp = _deg_sc(onesL, dst_a, zerosH)

    b1r, g1r, be1r = b1.reshape(1, H), g1.reshape(1, H), be1.reshape(1, H)
    b2r, g2r, be2r = b2.reshape(1, H), g2.reshape(1, H), be2.reshape(1, H)
    b3r, g3r, be3r = b3.reshape(1, H), g3.reshape(1, H), be3.reshape(1, H)

    y1, dis = _tcA(x, W1, degp)
    a1 = _agg_sc(y1, src_a, dst_a, zerosH)
    y2 = _tcB(a1, y1, dis, b1r, g1r, be1r, W2)
    a2 = _agg_sc(y2, src_a, dst_a, zerosH)
    y3 = _tcB(a2, y2, dis, b2r, g2r, be2r, W3)
    a3 = _agg_sc(y3, src_a, dst_a, zerosH)
    out = _tcC(a3, y3, dis, b3r, g3r, be3r,
               batch.reshape(1, N), fcW, fcb.reshape(1, O))
    return out
